# baseline (speedup 1.0000x reference)
import functools

import jax
import jax.numpy as jnp
from jax import lax
from jax.experimental import pallas as pl
from jax.experimental.pallas import tpu as pltpu
from jax.experimental.pallas import tpu_sc as plsc

VOCAB = 100000
EMB = 64
BATCH = 16384

NC = 2
NS = 16
NW = NC * NS
BPW = BATCH // NW
CH = 128
NCH = BPW // CH


def _sc_gather_body(user_h, item_h, tbl_h, ou, oi, idx_u, idx_i, rows, sem):
    c = lax.axis_index("c")
    s = lax.axis_index("s")
    wid = s * NC + c
    base = wid * BPW
    for j in range(NCH):
        pltpu.sync_copy(user_h.at[pl.ds(base + j * CH, CH)], idx_u.at[j])
        pltpu.sync_copy(item_h.at[pl.ds(base + j * CH, CH)], idx_i.at[j])
    for idx, out in ((idx_u, ou), (idx_i, oi)):
        cps = [pltpu.async_copy(tbl_h.at[idx.at[j]],
                                rows.at[pl.ds(j * CH, CH)], sem)
               for j in range(NCH)]
        for cp in cps:
            cp.wait()
        pltpu.sync_copy(rows, out.at[pl.ds(base, BPW)])


@functools.lru_cache(maxsize=1)
def _sc_gather():
    return pl.kernel(
        _sc_gather_body,
        out_type=tuple(jax.ShapeDtypeStruct((BATCH, 2 * EMB), jnp.float32)
                       for _ in range(2)),
        mesh=plsc.VectorSubcoreMesh(core_axis_name="c", subcore_axis_name="s",
                                    num_cores=NC, num_subcores=NS),
        scratch_types=[
            pltpu.VMEM((NCH, CH), jnp.int32),
            pltpu.VMEM((NCH, CH), jnp.int32),
            pltpu.VMEM((BPW, 2 * EMB), jnp.float32),
            pltpu.SemaphoreType.DMA,
        ],
        compiler_params=pltpu.CompilerParams(use_tc_tiling_on_sc=False),
    )


BLK = 2048


def _mlp_body(gu, gi, W1, b1, W2, b2, Wp, bp, out):
    h = jnp.concatenate((gu[:, :EMB], gi[:, EMB:]), axis=1)
    h = jnp.dot(h, W1[...], preferred_element_type=jnp.float32) + b1[...]
    h = jnp.maximum(h, 0.0)
    h = jnp.dot(h, W2[...], preferred_element_type=jnp.float32) + b2[...]
    h = jnp.maximum(h, 0.0)
    o = jnp.dot(h, Wp[...], preferred_element_type=jnp.float32) + bp[...]
    out[...] = jax.nn.sigmoid(o)


def _row_spec():
    return pl.BlockSpec((BLK, 2 * EMB), lambda i: (i, 0))


def _full_spec(shape):
    return pl.BlockSpec(shape, lambda i: tuple(0 for _ in shape))


_mlp = pl.pallas_call(
    _mlp_body,
    grid=(BATCH // BLK,),
    in_specs=[
        _row_spec(), _row_spec(),
        _full_spec((2 * EMB, 128)), _full_spec((1, 128)),
        _full_spec((128, 64)), _full_spec((1, 64)),
        _full_spec((64, 1)), _full_spec((1, 1)),
    ],
    out_specs=pl.BlockSpec((BLK, 1), lambda i: (i, 0)),
    out_shape=jax.ShapeDtypeStruct((BATCH, 1), jnp.float32),
)


@jax.jit
def kernel(x, su_emb, tu_emb, si_emb, ti_emb, W1, b1, W2, b2, Wp, bp):
    x = x.astype(jnp.int32)
    user = x[:, 0]
    item = x[:, 1]
    M = jnp.maximum(jnp.concatenate((su_emb, si_emb), axis=1),
                    jnp.concatenate((tu_emb, ti_emb), axis=1))
    gu, gi = _sc_gather()(user, item, M)
    out = _mlp(gu, gi,
               W1, b1.reshape(1, -1), W2, b2.reshape(1, -1),
               Wp, bp.reshape(1, 1))
    return out[:, 0]


# speedup vs baseline: 1.1920x; 1.1920x over previous
import functools

import jax
import jax.numpy as jnp
from jax import lax
from jax.experimental import pallas as pl
from jax.experimental.pallas import tpu as pltpu
from jax.experimental.pallas import tpu_sc as plsc

VOCAB = 100000
EMB = 64
BATCH = 16384

NC = 2
NS = 16
NW = NC * NS
BPW = BATCH // NW
CH = 128
NCH = BPW // CH


def _sc_gather_body(user_h, item_h, tbl_h, ou, oi, idx_u, idx_i, rows, sem):
    c = lax.axis_index("c")
    s = lax.axis_index("s")
    wid = s * NC + c
    base = wid * BPW
    for j in range(NCH):
        pltpu.sync_copy(user_h.at[pl.ds(base + j * CH, CH)], idx_u.at[j])
        pltpu.sync_copy(item_h.at[pl.ds(base + j * CH, CH)], idx_i.at[j])
    for idx, out in ((idx_u, ou), (idx_i, oi)):
        cps = [pltpu.async_copy(tbl_h.at[idx.at[j]],
                                rows.at[pl.ds(j * CH, CH)], sem)
               for j in range(NCH)]
        for cp in cps:
            cp.wait()
        pltpu.sync_copy(rows, out.at[pl.ds(base, BPW)])


@functools.lru_cache(maxsize=1)
def _sc_gather():
    return pl.kernel(
        _sc_gather_body,
        out_type=tuple(jax.ShapeDtypeStruct((BATCH, 2 * EMB), jnp.float32)
                       for _ in range(2)),
        mesh=plsc.VectorSubcoreMesh(core_axis_name="c", subcore_axis_name="s",
                                    num_cores=NC, num_subcores=NS),
        scratch_types=[
            pltpu.VMEM((NCH, CH), jnp.int32),
            pltpu.VMEM((NCH, CH), jnp.int32),
            pltpu.VMEM((BPW, 2 * EMB), jnp.float32),
            pltpu.SemaphoreType.DMA,
        ],
        compiler_params=pltpu.CompilerParams(use_tc_tiling_on_sc=False),
    )


BLK = 2048


def _mlp_body(gu, gi, W1, b1, W2, b2, Wp, bp, out):
    h = jnp.concatenate((gu[:, :EMB], gi[:, EMB:]), axis=1)
    h = jnp.dot(h, W1[...], preferred_element_type=jnp.float32) + b1[...]
    h = jnp.maximum(h, 0.0)
    h = jnp.dot(h, W2[...], preferred_element_type=jnp.float32) + b2[...]
    h = jnp.maximum(h, 0.0)
    o = jnp.dot(h, Wp[...], preferred_element_type=jnp.float32) + bp[...]
    out[...] = jax.nn.sigmoid(o)


def _row_spec():
    return pl.BlockSpec((BLK, 2 * EMB), lambda i: (i, 0))


def _full_spec(shape):
    return pl.BlockSpec(shape, lambda i: tuple(0 for _ in shape))


_mlp = pl.pallas_call(
    _mlp_body,
    grid=(BATCH // BLK,),
    in_specs=[
        _row_spec(), _row_spec(),
        _full_spec((2 * EMB, 128)), _full_spec((1, 128)),
        _full_spec((128, 64)), _full_spec((1, 64)),
        _full_spec((64, 1)), _full_spec((1, 1)),
    ],
    out_specs=pl.BlockSpec((BLK, 1), lambda i: (i, 0)),
    out_shape=jax.ShapeDtypeStruct((BATCH, 1), jnp.float32),
)


@jax.jit
def kernel(x, su_emb, tu_emb, si_emb, ti_emb, W1, b1, W2, b2, Wp, bp):
    x = x.astype(jnp.int32)
    user = x[:, 0]
    item = x[:, 1]
    M = jnp.concatenate((jnp.maximum(su_emb, tu_emb),
                         jnp.maximum(si_emb, ti_emb)), axis=1)
    gu, gi = _sc_gather()(user, item, M)
    out = _mlp(gu, gi,
               W1, b1.reshape(1, -1), W2, b2.reshape(1, -1),
               Wp, bp.reshape(1, 1))
    return out[:, 0]
